# Initial kernel scaffold; baseline (speedup 1.0000x reference)
import jax
import jax.numpy as jnp
from jax.experimental import pallas as pl


def kernel(x, edge_index, DDI_features, W1, b1, W2, b2, Wf1, bf1, Wf2, bf2, Wf3, bf3):
    raise NotImplementedError("write your pallas kernel here")


# speedup vs baseline: 12.4519x; 12.4519x over previous
import functools

import jax
import jax.numpy as jnp
from jax import lax
from jax.experimental import pallas as pl
from jax.experimental.pallas import tpu as pltpu
from jax.experimental.pallas import tpu_sc as plsc

N = 10000
E = 320000
D = 128
NC = 2
NS = 16
NW = NC * NS
K = 128
N_ACC = 10112
E_PAD = 323584
BM = 2000

_mesh = functools.partial(
    plsc.VectorSubcoreMesh, core_axis_name="c", subcore_axis_name="s")


def _deg_body(colp_hbm, ones_hbm, zeros_hbm, out_hbm, acc, idx_v, ones_v):
    c = lax.axis_index("c")
    s = lax.axis_index("s")
    rows = N_ACC // NS
    pltpu.sync_copy(zeros_hbm.at[pl.ds(s * rows, rows)],
                    acc.at[pl.ds(s * rows, rows)])
    pltpu.sync_copy(ones_hbm, ones_v)
    plsc.subcore_barrier()
    per_tile = E_PAD // NW
    base = (c * NS + s) * per_tile

    def step(k, carry):
        pltpu.sync_copy(colp_hbm.at[pl.ds(base + k * K, K)], idx_v)
        pltpu.sync_copy(ones_v, acc.at[idx_v], add=True)
        return carry

    lax.fori_loop(0, per_tile // K, step, 0)
    plsc.subcore_barrier()
    pltpu.sync_copy(acc.at[pl.ds(s * rows, rows)],
                    out_hbm.at[c, pl.ds(s * rows, rows)])


def _deg_partials(colp, ones8, zeros8):
    return pl.kernel(
        _deg_body,
        out_type=jax.ShapeDtypeStruct((NC, N_ACC, 8), jnp.float32),
        mesh=_mesh(),
        scratch_types=[
            pltpu.VMEM_SHARED((N_ACC, 8), jnp.float32),
            pltpu.VMEM((K,), jnp.int32),
            pltpu.VMEM((K, 8), jnp.float32),
        ],
    )(colp, ones8, zeros8)


def _scatter_body(split_all, table_hbm, rowp_hbm, colp_hbm, zeros_hbm,
                  out_hbm, acc, ridx_v, cidx_v, gbuf, sem):
    c = lax.axis_index("c")
    s = lax.axis_index("s")
    rows = N_ACC // NS
    pltpu.sync_copy(zeros_hbm.at[pl.ds(s * rows, rows)],
                    acc.at[pl.ds(s * rows, rows)])
    plsc.subcore_barrier()
    if split_all:
        per_tile = E_PAD // NW
        base = (c * NS + s) * per_tile
        tix = 0
    else:
        per_tile = E_PAD // NS
        base = s * per_tile
        tix = c

    def step(k, carry):
        eb = base + k * K
        pltpu.sync_copy(rowp_hbm.at[pl.ds(eb, K)], ridx_v)
        pltpu.sync_copy(colp_hbm.at[pl.ds(eb, K)], cidx_v)
        pltpu.async_copy(table_hbm.at[tix].at[ridx_v], gbuf, sem).wait()
        pltpu.sync_copy(gbuf, acc.at[cidx_v], add=True)
        return carry

    lax.fori_loop(0, per_tile // K, step, 0)
    plsc.subcore_barrier()
    pltpu.sync_copy(acc.at[pl.ds(s * rows, rows)],
                    out_hbm.at[c, pl.ds(s * rows, rows)])


def _scatter_partials(table, rowp, colp, zerosD, split_all):
    return pl.kernel(
        functools.partial(_scatter_body, split_all),
        out_type=jax.ShapeDtypeStruct((NC, N_ACC, D), jnp.float32),
        mesh=_mesh(),
        scratch_types=[
            pltpu.VMEM_SHARED((N_ACC, D), jnp.float32),
            pltpu.VMEM((K,), jnp.int32),
            pltpu.VMEM((K,), jnp.int32),
            pltpu.VMEM((K, D), jnp.float32),
            pltpu.SemaphoreType.DMA,
        ],
    )(table, rowp, colp, zerosD)


def _dinv(dega_ref, degb_ref):
    deg = dega_ref[:, 0:1] + degb_ref[:, 0:1] + 1.0
    return lax.rsqrt(deg)


def _h1p_body(x_ref, w_ref, dega_ref, degb_ref, o_ref):
    dinv = _dinv(dega_ref, degb_ref)
    o_ref[...] = jnp.dot(
        x_ref[...], w_ref[...], preferred_element_type=jnp.float32) * dinv


def _h2p_body(s1a_ref, s1b_ref, h1p_ref, dega_ref, degb_ref, b1_ref, w2_ref,
              o_ref):
    dinv = _dinv(dega_ref, degb_ref)
    t = jnp.maximum(
        dinv * (s1a_ref[0] + s1b_ref[0] + h1p_ref[...]) + b1_ref[...],
        0.0)
    o_ref[0] = jnp.dot(
        t, w2_ref[...], preferred_element_type=jnp.float32) * dinv


def _ppi_body(s2_ref, h2p_ref, dega_ref, degb_ref, b2_ref, o_ref):
    dinv = _dinv(dega_ref, degb_ref)
    o_ref[...] = jnp.maximum(
        dinv * (s2_ref[0] + h2p_ref[0]) + b2_ref[0], 0.0)


def _ddi_body(f_ref, w1_ref, b1_ref, w2_ref, b2_ref, w3_ref, b3_ref, o_ref):
    t = jnp.maximum(
        jnp.dot(f_ref[...], w1_ref[...], preferred_element_type=jnp.float32)
        + b1_ref[...], 0.0)
    t = jnp.maximum(
        jnp.dot(t, w2_ref[...], preferred_element_type=jnp.float32)
        + b2_ref[...], 0.0)
    o_ref[...] = jnp.maximum(
        jnp.dot(t, w3_ref[...], preferred_element_type=jnp.float32)
        + b3_ref[...], 0.0)


def kernel(x, edge_index, DDI_features, W1, b1, W2, b2,
           Wf1, bf1, Wf2, bf2, Wf3, bf3):
    row = edge_index[0]
    col = edge_index[1]
    pad = E_PAD - E
    pad_i = jnp.arange(pad, dtype=jnp.int32)
    rowp = jnp.concatenate([row, pad_i % N])
    colp = jnp.concatenate([col, N + pad_i % (N_ACC - N)])
    zeros8 = jnp.zeros((N_ACC, 8), jnp.float32)
    ones8 = jnp.ones((K, 8), jnp.float32)
    zerosD = jnp.zeros((N_ACC, D), jnp.float32)

    deg = _deg_partials(colp, ones8, zeros8)
    dega, degb = deg[0], deg[1]

    grid5 = 5
    h1p = pl.pallas_call(
        _h1p_body,
        grid=(grid5,),
        in_specs=[
            pl.BlockSpec((BM, D), lambda i: (i, 0)),
            pl.BlockSpec((D, D), lambda i: (0, 0)),
            pl.BlockSpec((BM, 8), lambda i: (i, 0)),
            pl.BlockSpec((BM, 8), lambda i: (i, 0)),
        ],
        out_specs=pl.BlockSpec((BM, D), lambda i: (i, 0)),
        out_shape=jax.ShapeDtypeStruct((N, D), jnp.float32),
    )(x, W1, dega, degb)

    s1 = _scatter_partials(h1p[None], rowp, colp, zerosD, split_all=True)

    b1r = b1.reshape(1, D)
    h2p = pl.pallas_call(
        _h2p_body,
        grid=(2, grid5),
        in_specs=[
            pl.BlockSpec((1, BM, D), lambda j, i: (0, i, 0)),
            pl.BlockSpec((1, BM, D), lambda j, i: (0, i, 0)),
            pl.BlockSpec((BM, D), lambda j, i: (i, 0)),
            pl.BlockSpec((BM, 8), lambda j, i: (i, 0)),
            pl.BlockSpec((BM, 8), lambda j, i: (i, 0)),
            pl.BlockSpec((1, D), lambda j, i: (0, 0)),
            pl.BlockSpec((D, D), lambda j, i: (0, j)),
        ],
        out_specs=pl.BlockSpec((1, BM, D), lambda j, i: (j, i, 0)),
        out_shape=jax.ShapeDtypeStruct((NC, N, D), jnp.float32),
    )(s1[0:1], s1[1:2], h1p, dega, degb, b1r, W2)

    s2 = _scatter_partials(h2p, rowp, colp, zerosD, split_all=False)

    b2r = b2.reshape(NC, 1, D)
    ppi = pl.pallas_call(
        _ppi_body,
        grid=(2, grid5),
        in_specs=[
            pl.BlockSpec((1, BM, D), lambda j, i: (j, i, 0)),
            pl.BlockSpec((1, BM, D), lambda j, i: (j, i, 0)),
            pl.BlockSpec((BM, 8), lambda j, i: (i, 0)),
            pl.BlockSpec((BM, 8), lambda j, i: (i, 0)),
            pl.BlockSpec((1, 1, D), lambda j, i: (j, 0, 0)),
        ],
        out_specs=pl.BlockSpec((BM, D), lambda j, i: (i, j)),
        out_shape=jax.ShapeDtypeStruct((N, NC * D), jnp.float32),
    )(s2, h2p, dega, degb, b2r)

    B, DDI_DIM = DDI_features.shape
    ddi = pl.pallas_call(
        _ddi_body,
        out_shape=jax.ShapeDtypeStruct((B, 1), jnp.float32),
    )(DDI_features, Wf1, bf1.reshape(1, 64), Wf2, bf2.reshape(1, 16),
      Wf3, bf3.reshape(1, 1))

    return (ppi, ddi)


# speedup vs baseline: 21.2441x; 1.7061x over previous
import functools

import jax
import jax.numpy as jnp
from jax import lax
from jax.experimental import pallas as pl
from jax.experimental.pallas import tpu as pltpu
from jax.experimental.pallas import tpu_sc as plsc

N = 10000
E = 320000
D = 128
NC = 2
NS = 16
NW = NC * NS
K = 128
CB = 1
N_ACC = 10112
E_PAD = 327680
NCHUNK = E_PAD // K
BM = 2000

_mesh = functools.partial(
    plsc.VectorSubcoreMesh, core_axis_name="c", subcore_axis_name="s")


def _deg_body(colp_hbm, ones_hbm, zeros_hbm, out_hbm, acc, idx_v, ones_v):
    c = lax.axis_index("c")
    s = lax.axis_index("s")
    rows = N_ACC // NS
    pltpu.sync_copy(zeros_hbm.at[pl.ds(s * rows, rows)],
                    acc.at[pl.ds(s * rows, rows)])
    pltpu.sync_copy(ones_hbm, ones_v)
    plsc.subcore_barrier()
    per_tile = E_PAD // NW
    base = (c * NS + s) * per_tile

    def step(k, carry):
        pltpu.sync_copy(colp_hbm.at[pl.ds(base + k * K, K)], idx_v)
        pltpu.sync_copy(ones_v, acc.at[idx_v], add=True)
        return carry

    lax.fori_loop(0, per_tile // K, step, 0)
    plsc.subcore_barrier()
    pltpu.sync_copy(acc.at[pl.ds(s * rows, rows)],
                    out_hbm.at[c, pl.ds(s * rows, rows)])


def _deg_partials(colp, ones8, zeros8):
    return pl.kernel(
        _deg_body,
        out_type=jax.ShapeDtypeStruct((NC, N_ACC, 8), jnp.float32),
        mesh=_mesh(),
        scratch_types=[
            pltpu.VMEM_SHARED((N_ACC, 8), jnp.float32),
            pltpu.VMEM((K,), jnp.int32),
            pltpu.VMEM((K, 8), jnp.float32),
        ],
    )(colp, ones8, zeros8)


def _scatter_body(split_all, table_hbm, eidx_hbm, zeros_hbm,
                  out_hbm, acc, ibuf, gbuf, rs0, rs1, ws0, ws1):
    c = lax.axis_index("c")
    s = lax.axis_index("s")
    rows = N_ACC // NS
    pltpu.sync_copy(zeros_hbm.at[pl.ds(s * rows, rows)],
                    acc.at[pl.ds(s * rows, rows)])
    plsc.subcore_barrier()
    if split_all:
        n_chunks = E_PAD // NW // K
        cb0 = (c * NS + s) * n_chunks
        tix = 0
    else:
        n_chunks = E_PAD // NS // K
        cb0 = s * n_chunks
        tix = c
    G = n_chunks // CB
    rsem = (rs0, rs1)
    wsem = (ws0, ws1)

    def g_desc(slot, j):
        return pltpu.make_async_copy(
            table_hbm.at[tix].at[ibuf.at[slot, j, 0]],
            gbuf.at[slot, j], rsem[slot])

    def s_desc(slot, j):
        return pltpu.make_async_copy(
            gbuf.at[slot, j], acc.at[ibuf.at[slot, j, 1]], wsem[slot])

    def load(g, slot, first=False):
        if not first:
            for j in range(CB):
                s_desc(slot, j).wait()
        pltpu.sync_copy(eidx_hbm.at[pl.ds(cb0 + g * CB, CB)], ibuf.at[slot])
        for j in range(CB):
            g_desc(slot, j).start()

    def flush(slot):
        for j in range(CB):
            g_desc(slot, j).wait()
        for j in range(CB):
            s_desc(slot, j).start(add=True)

    load(0, 0, first=True)
    load(1, 1, first=True)
    flush(0)

    def body(u, carry):
        g0 = 2 * u
        load(g0, 0)
        flush(1)
        load(g0 + 1, 1)
        flush(0)
        return carry

    lax.fori_loop(1, G // 2, body, 0)
    flush(1)
    for slot in range(2):
        for j in range(CB):
            s_desc(slot, j).wait()
    plsc.subcore_barrier()
    pltpu.sync_copy(acc.at[pl.ds(s * rows, rows)],
                    out_hbm.at[c, pl.ds(s * rows, rows)])


def _scatter_partials(table, eidx, zerosD, split_all):
    return pl.kernel(
        functools.partial(_scatter_body, split_all),
        out_type=jax.ShapeDtypeStruct((NC, N_ACC, D), jnp.float32),
        mesh=_mesh(),
        scratch_types=[
            pltpu.VMEM_SHARED((N_ACC, D), jnp.float32),
            pltpu.VMEM((2, CB, 2, K), jnp.int32),
            pltpu.VMEM((2, CB, K, D), jnp.float32),
            pltpu.SemaphoreType.DMA,
            pltpu.SemaphoreType.DMA,
            pltpu.SemaphoreType.DMA,
            pltpu.SemaphoreType.DMA,
        ],
    )(table, eidx, zerosD)


def _dinv(dega_ref, degb_ref):
    deg = dega_ref[:, 0:1] + degb_ref[:, 0:1] + 1.0
    return lax.rsqrt(deg)


def _h1p_body(x_ref, w_ref, dega_ref, degb_ref, o_ref):
    dinv = _dinv(dega_ref, degb_ref)
    o_ref[...] = jnp.dot(
        x_ref[...], w_ref[...], preferred_element_type=jnp.float32) * dinv


def _h2p_body(s1a_ref, s1b_ref, h1p_ref, dega_ref, degb_ref, b1_ref, w2_ref,
              o_ref):
    dinv = _dinv(dega_ref, degb_ref)
    t = jnp.maximum(
        dinv * (s1a_ref[0] + s1b_ref[0] + h1p_ref[...]) + b1_ref[...],
        0.0)
    o_ref[0] = jnp.dot(
        t, w2_ref[...], preferred_element_type=jnp.float32) * dinv


def _ppi_body(s2_ref, h2p_ref, dega_ref, degb_ref, b2_ref, o_ref):
    dinv = _dinv(dega_ref, degb_ref)
    o_ref[...] = jnp.maximum(
        dinv * (s2_ref[0] + h2p_ref[0]) + b2_ref[0], 0.0)


def _ddi_body(f_ref, w1_ref, b1_ref, w2_ref, b2_ref, w3_ref, b3_ref, o_ref):
    t = jnp.maximum(
        jnp.dot(f_ref[...], w1_ref[...], preferred_element_type=jnp.float32)
        + b1_ref[...], 0.0)
    t = jnp.maximum(
        jnp.dot(t, w2_ref[...], preferred_element_type=jnp.float32)
        + b2_ref[...], 0.0)
    o_ref[...] = jnp.maximum(
        jnp.dot(t, w3_ref[...], preferred_element_type=jnp.float32)
        + b3_ref[...], 0.0)


def kernel(x, edge_index, DDI_features, W1, b1, W2, b2,
           Wf1, bf1, Wf2, bf2, Wf3, bf3):
    row = edge_index[0]
    col = edge_index[1]
    pad = E_PAD - E
    pad_i = jnp.arange(pad, dtype=jnp.int32)
    rowp = jnp.concatenate([row, pad_i % N])
    colp = jnp.concatenate([col, N + pad_i % (N_ACC - N)])
    eidx = jnp.stack(
        [rowp.reshape(NCHUNK, K), colp.reshape(NCHUNK, K)], axis=1)
    zeros8 = jnp.zeros((N_ACC, 8), jnp.float32)
    ones8 = jnp.ones((K, 8), jnp.float32)
    zerosD = jnp.zeros((N_ACC, D), jnp.float32)

    deg = _deg_partials(colp, ones8, zeros8)
    dega, degb = deg[0], deg[1]

    grid5 = 5
    h1p = pl.pallas_call(
        _h1p_body,
        grid=(grid5,),
        in_specs=[
            pl.BlockSpec((BM, D), lambda i: (i, 0)),
            pl.BlockSpec((D, D), lambda i: (0, 0)),
            pl.BlockSpec((BM, 8), lambda i: (i, 0)),
            pl.BlockSpec((BM, 8), lambda i: (i, 0)),
        ],
        out_specs=pl.BlockSpec((BM, D), lambda i: (i, 0)),
        out_shape=jax.ShapeDtypeStruct((N, D), jnp.float32),
    )(x, W1, dega, degb)

    s1 = _scatter_partials(h1p[None], eidx, zerosD, split_all=True)

    b1r = b1.reshape(1, D)
    h2p = pl.pallas_call(
        _h2p_body,
        grid=(2, grid5),
        in_specs=[
            pl.BlockSpec((1, BM, D), lambda j, i: (0, i, 0)),
            pl.BlockSpec((1, BM, D), lambda j, i: (0, i, 0)),
            pl.BlockSpec((BM, D), lambda j, i: (i, 0)),
            pl.BlockSpec((BM, 8), lambda j, i: (i, 0)),
            pl.BlockSpec((BM, 8), lambda j, i: (i, 0)),
            pl.BlockSpec((1, D), lambda j, i: (0, 0)),
            pl.BlockSpec((D, D), lambda j, i: (0, j)),
        ],
        out_specs=pl.BlockSpec((1, BM, D), lambda j, i: (j, i, 0)),
        out_shape=jax.ShapeDtypeStruct((NC, N, D), jnp.float32),
    )(s1[0:1], s1[1:2], h1p, dega, degb, b1r, W2)

    s2 = _scatter_partials(h2p, eidx, zerosD, split_all=False)

    b2r = b2.reshape(NC, 1, D)
    ppi = pl.pallas_call(
        _ppi_body,
        grid=(2, grid5),
        in_specs=[
            pl.BlockSpec((1, BM, D), lambda j, i: (j, i, 0)),
            pl.BlockSpec((1, BM, D), lambda j, i: (j, i, 0)),
            pl.BlockSpec((BM, 8), lambda j, i: (i, 0)),
            pl.BlockSpec((BM, 8), lambda j, i: (i, 0)),
            pl.BlockSpec((1, 1, D), lambda j, i: (j, 0, 0)),
        ],
        out_specs=pl.BlockSpec((BM, D), lambda j, i: (i, j)),
        out_shape=jax.ShapeDtypeStruct((N, NC * D), jnp.float32),
    )(s2, h2p, dega, degb, b2r)

    B, DDI_DIM = DDI_features.shape
    ddi = pl.pallas_call(
        _ddi_body,
        out_shape=jax.ShapeDtypeStruct((B, 1), jnp.float32),
    )(DDI_features, Wf1, bf1.reshape(1, 64), Wf2, bf2.reshape(1, 16),
      Wf3, bf3.reshape(1, 1))

    return (ppi, ddi)


# speedup vs baseline: 23.0904x; 1.0869x over previous
import functools

import jax
import jax.numpy as jnp
from jax import lax
from jax.experimental import pallas as pl
from jax.experimental.pallas import tpu as pltpu
from jax.experimental.pallas import tpu_sc as plsc

N = 10000
E = 320000
D = 128
NC = 2
NS = 16
NW = NC * NS
K = 64
N_ACC = 10112
E_PAD = 327680
NCHUNK = E_PAD // K
BM = 2000

_mesh = functools.partial(
    plsc.VectorSubcoreMesh, core_axis_name="c", subcore_axis_name="s")


def _deg_body(eidx_hbm, ones_hbm, zeros_hbm, out_hbm,
              acc, ibuf, ones_v, is0, is1, is2, is3, is4, is5, is6, is7,
              ws0, ws1, ws2, ws3):
    c = lax.axis_index("c")
    s = lax.axis_index("s")
    rows = N_ACC // NS
    pltpu.sync_copy(zeros_hbm.at[pl.ds(s * rows, rows)],
                    acc.at[pl.ds(s * rows, rows)])
    pltpu.sync_copy(ones_hbm, ones_v)
    plsc.subcore_barrier()
    n = E_PAD // NW // K
    cb0 = (c * NS + s) * n
    isem = (is0, is1, is2, is3, is4, is5, is6, is7)
    ws = (ws0, ws1, ws2, ws3)

    def i_desc(k, j8):
        return pltpu.make_async_copy(
            eidx_hbm.at[lax.rem(cb0 + k, NCHUNK)], ibuf.at[j8], isem[j8])

    def s_desc(j8, j4):
        return pltpu.make_async_copy(
            ones_v, acc.at[ibuf.at[j8, 1]], ws[j4])

    def pos(k, j8, wait_w):
        j4 = j8 % 4
        if wait_w:
            s_desc((j8 + 4) % 8, j4).wait()
        i_desc(k, j8).wait()
        s_desc(j8, j4).start(add=True)
        i_desc(k + 4, (j8 + 4) % 8).start()

    for j in range(4):
        i_desc(j, j).start()
    for k in range(8):
        pos(k, k, wait_w=(k >= 4))

    def body(v, carry):
        k0 = 8 * v
        for j in range(8):
            pos(k0 + j, j, wait_w=True)
        return carry

    lax.fori_loop(1, n // 8, body, 0)
    for j in range(4):
        s_desc(j, j % 4).wait()
    for j in range(4):
        i_desc(0, j).wait()
    plsc.subcore_barrier()
    pltpu.sync_copy(acc.at[pl.ds(s * rows, rows)],
                    out_hbm.at[c, pl.ds(s * rows, rows)])


def _deg_partials(eidx, onesD, zerosD):
    return pl.kernel(
        _deg_body,
        out_type=jax.ShapeDtypeStruct((NC, N_ACC, D), jnp.float32),
        mesh=_mesh(),
        scratch_types=[
            pltpu.VMEM_SHARED((N_ACC, D), jnp.float32),
            pltpu.VMEM((8, 2, K), jnp.int32),
            pltpu.VMEM((K, D), jnp.float32),
        ] + [pltpu.SemaphoreType.DMA] * 12,
    )(eidx, onesD, zerosD)


def _scatter_body(split_all, table_hbm, eidx_hbm, zeros_hbm, out_hbm,
                  acc, ibuf, gbuf, is0, is1, is2, is3, is4, is5, is6, is7,
                  rs0, rs1, rs2, rs3, ws0, ws1, ws2, ws3):
    c = lax.axis_index("c")
    s = lax.axis_index("s")
    rows = N_ACC // NS
    pltpu.sync_copy(zeros_hbm.at[pl.ds(s * rows, rows)],
                    acc.at[pl.ds(s * rows, rows)])
    plsc.subcore_barrier()
    if split_all:
        n = E_PAD // NW // K
        cb0 = (c * NS + s) * n
        tix = 0
    else:
        n = E_PAD // NS // K
        cb0 = s * n
        tix = c
    isem = (is0, is1, is2, is3, is4, is5, is6, is7)
    rs = (rs0, rs1, rs2, rs3)
    ws = (ws0, ws1, ws2, ws3)

    def i_desc(k, j8):
        return pltpu.make_async_copy(
            eidx_hbm.at[lax.rem(cb0 + k, NCHUNK)], ibuf.at[j8], isem[j8])

    def g_desc(j8):
        j4 = j8 % 4
        return pltpu.make_async_copy(
            table_hbm.at[tix].at[ibuf.at[j8, 0]], gbuf.at[j4], rs[j4])

    def s_desc(j8):
        j4 = j8 % 4
        return pltpu.make_async_copy(
            gbuf.at[j4], acc.at[ibuf.at[j8, 1]], ws[j4])

    def pos(k, j8, wait_w, do_scat):
        if wait_w:
            s_desc((j8 + 4) % 8).wait()
        i_desc(k, j8).wait()
        g_desc(j8).start()
        i_desc(k + 4, (j8 + 4) % 8).start()
        if do_scat:
            g_desc((j8 + 6) % 8).wait()
            s_desc((j8 + 6) % 8).start(add=True)

    for j in range(4):
        i_desc(j, j).start()
    for k in range(8):
        pos(k, k, wait_w=(k >= 4), do_scat=(k >= 2))

    def body(v, carry):
        k0 = 8 * v
        for j in range(8):
            pos(k0 + j, j, wait_w=True, do_scat=True)
        return carry

    lax.fori_loop(1, n // 8, body, 0)
    for j8 in (6, 7):
        g_desc(j8).wait()
        s_desc(j8).start(add=True)
    for j in range(4):
        s_desc(j).wait()
        i_desc(0, j).wait()
    plsc.subcore_barrier()
    pltpu.sync_copy(acc.at[pl.ds(s * rows, rows)],
                    out_hbm.at[c, pl.ds(s * rows, rows)])


def _scatter_partials(table, eidx, zerosD, split_all):
    return pl.kernel(
        functools.partial(_scatter_body, split_all),
        out_type=jax.ShapeDtypeStruct((NC, N_ACC, D), jnp.float32),
        mesh=_mesh(),
        scratch_types=[
            pltpu.VMEM_SHARED((N_ACC, D), jnp.float32),
            pltpu.VMEM((8, 2, K), jnp.int32),
            pltpu.VMEM((4, K, D), jnp.float32),
        ] + [pltpu.SemaphoreType.DMA] * 16,
    )(table, eidx, zerosD)


def _dinv(dega_ref, degb_ref):
    deg = dega_ref[:, 0:1] + degb_ref[:, 0:1] + 1.0
    return lax.rsqrt(deg)


def _h1p_body(x_ref, w_ref, dega_ref, degb_ref, o_ref):
    dinv = _dinv(dega_ref, degb_ref)
    o_ref[...] = jnp.dot(
        x_ref[...], w_ref[...], preferred_element_type=jnp.float32) * dinv


def _h2p_body(s1a_ref, s1b_ref, h1p_ref, dega_ref, degb_ref, b1_ref, w2_ref,
              o_ref):
    dinv = _dinv(dega_ref, degb_ref)
    t = jnp.maximum(
        dinv * (s1a_ref[0] + s1b_ref[0] + h1p_ref[...]) + b1_ref[...],
        0.0)
    o_ref[0] = jnp.dot(
        t, w2_ref[...], preferred_element_type=jnp.float32) * dinv


def _ppi_body(s2_ref, h2p_ref, dega_ref, degb_ref, b2_ref, o_ref):
    dinv = _dinv(dega_ref, degb_ref)
    o_ref[...] = jnp.maximum(
        dinv * (s2_ref[0] + h2p_ref[0]) + b2_ref[0], 0.0)


def _ddi_body(f_ref, w1_ref, b1_ref, w2_ref, b2_ref, w3_ref, b3_ref, o_ref):
    t = jnp.maximum(
        jnp.dot(f_ref[...], w1_ref[...], preferred_element_type=jnp.float32)
        + b1_ref[...], 0.0)
    t = jnp.maximum(
        jnp.dot(t, w2_ref[...], preferred_element_type=jnp.float32)
        + b2_ref[...], 0.0)
    o_ref[...] = jnp.maximum(
        jnp.dot(t, w3_ref[...], preferred_element_type=jnp.float32)
        + b3_ref[...], 0.0)


def kernel(x, edge_index, DDI_features, W1, b1, W2, b2,
           Wf1, bf1, Wf2, bf2, Wf3, bf3):
    row = edge_index[0]
    col = edge_index[1]
    pad = E_PAD - E
    pad_i = jnp.arange(pad, dtype=jnp.int32)
    rowp = jnp.concatenate([row, pad_i % N])
    colp = jnp.concatenate([col, N + pad_i % (N_ACC - N)])
    eidx = jnp.stack(
        [rowp.reshape(NCHUNK, K), colp.reshape(NCHUNK, K)], axis=1)
    onesD = jnp.ones((K, D), jnp.float32)
    zerosD = jnp.zeros((N_ACC, D), jnp.float32)

    deg = _deg_partials(eidx, onesD, zerosD)
    dega, degb = deg[0], deg[1]

    grid5 = 5
    h1p = pl.pallas_call(
        _h1p_body,
        grid=(grid5,),
        in_specs=[
            pl.BlockSpec((BM, D), lambda i: (i, 0)),
            pl.BlockSpec((D, D), lambda i: (0, 0)),
            pl.BlockSpec((BM, D), lambda i: (i, 0)),
            pl.BlockSpec((BM, D), lambda i: (i, 0)),
        ],
        out_specs=pl.BlockSpec((BM, D), lambda i: (i, 0)),
        out_shape=jax.ShapeDtypeStruct((N, D), jnp.float32),
    )(x, W1, dega, degb)

    s1 = _scatter_partials(h1p[None], eidx, zerosD, split_all=True)

    b1r = b1.reshape(1, D)
    h2p = pl.pallas_call(
        _h2p_body,
        grid=(2, grid5),
        in_specs=[
            pl.BlockSpec((1, BM, D), lambda j, i: (0, i, 0)),
            pl.BlockSpec((1, BM, D), lambda j, i: (0, i, 0)),
            pl.BlockSpec((BM, D), lambda j, i: (i, 0)),
            pl.BlockSpec((BM, D), lambda j, i: (i, 0)),
            pl.BlockSpec((BM, D), lambda j, i: (i, 0)),
            pl.BlockSpec((1, D), lambda j, i: (0, 0)),
            pl.BlockSpec((D, D), lambda j, i: (0, j)),
        ],
        out_specs=pl.BlockSpec((1, BM, D), lambda j, i: (j, i, 0)),
        out_shape=jax.ShapeDtypeStruct((NC, N, D), jnp.float32),
    )(s1[0:1], s1[1:2], h1p, dega, degb, b1r, W2)

    s2 = _scatter_partials(h2p, eidx, zerosD, split_all=False)

    b2r = b2.reshape(NC, 1, D)
    ppi = pl.pallas_call(
        _ppi_body,
        grid=(2, grid5),
        in_specs=[
            pl.BlockSpec((1, BM, D), lambda j, i: (j, i, 0)),
            pl.BlockSpec((1, BM, D), lambda j, i: (j, i, 0)),
            pl.BlockSpec((BM, D), lambda j, i: (i, 0)),
            pl.BlockSpec((BM, D), lambda j, i: (i, 0)),
            pl.BlockSpec((1, 1, D), lambda j, i: (j, 0, 0)),
        ],
        out_specs=pl.BlockSpec((BM, D), lambda j, i: (i, j)),
        out_shape=jax.ShapeDtypeStruct((N, NC * D), jnp.float32),
    )(s2, h2p, dega, degb, b2r)

    B, DDI_DIM = DDI_features.shape
    ddi = pl.pallas_call(
        _ddi_body,
        out_shape=jax.ShapeDtypeStruct((B, 1), jnp.float32),
    )(DDI_features, Wf1, bf1.reshape(1, 64), Wf2, bf2.reshape(1, 16),
      Wf3, bf3.reshape(1, 1))

    return (ppi, ddi)


# speedup vs baseline: 23.1130x; 1.0010x over previous
import functools

import jax
import jax.numpy as jnp
from jax import lax
from jax.experimental import pallas as pl
from jax.experimental.pallas import tpu as pltpu
from jax.experimental.pallas import tpu_sc as plsc

N = 10000
E = 320000
D = 128
NC = 2
NS = 16
NW = NC * NS
K = 64
N_ACC = 10112
E_PAD = 327680
NCHUNK = E_PAD // K
BM = 2000

_mesh = functools.partial(
    plsc.VectorSubcoreMesh, core_axis_name="c", subcore_axis_name="s")


def _deg_body(eidx_hbm, ones_hbm, zeros_hbm, out_hbm,
              acc, ibuf, ones_v, is0, is1, is2, is3, is4, is5, is6, is7,
              ws0, ws1, ws2, ws3):
    c = lax.axis_index("c")
    s = lax.axis_index("s")
    rows = N_ACC // NS
    pltpu.sync_copy(zeros_hbm.at[pl.ds(s * rows, rows)],
                    acc.at[pl.ds(s * rows, rows)])
    pltpu.sync_copy(ones_hbm, ones_v)
    plsc.subcore_barrier()
    n = E_PAD // NW // K
    cb0 = (c * NS + s) * n
    isem = (is0, is1, is2, is3, is4, is5, is6, is7)
    ws = (ws0, ws1, ws2, ws3)

    def i_desc(k, j8):
        return pltpu.make_async_copy(
            eidx_hbm.at[lax.rem(cb0 + k, NCHUNK)], ibuf.at[j8], isem[j8])

    def s_desc(j8, j4):
        return pltpu.make_async_copy(
            ones_v, acc.at[ibuf.at[j8, 1]], ws[j4])

    def pos(k, j8, wait_w):
        j4 = j8 % 4
        if wait_w:
            s_desc((j8 + 4) % 8, j4).wait()
        i_desc(k, j8).wait()
        s_desc(j8, j4).start(add=True)
        i_desc(k + 4, (j8 + 4) % 8).start()

    for j in range(4):
        i_desc(j, j).start()
    for k in range(8):
        pos(k, k, wait_w=(k >= 4))

    def body(v, carry):
        k0 = 8 * v
        for j in range(8):
            pos(k0 + j, j, wait_w=True)
        return carry

    lax.fori_loop(1, n // 8, body, 0)
    for j in range(4):
        s_desc(j, j % 4).wait()
    for j in range(4):
        i_desc(0, j).wait()
    plsc.subcore_barrier()
    pltpu.sync_copy(acc.at[pl.ds(s * rows, rows)],
                    out_hbm.at[c, pl.ds(s * rows, rows)])


def _deg_partials(eidx, onesD, zerosD):
    return pl.kernel(
        _deg_body,
        out_type=jax.ShapeDtypeStruct((NC, N_ACC, D), jnp.float32),
        mesh=_mesh(),
        scratch_types=[
            pltpu.VMEM_SHARED((N_ACC, D), jnp.float32),
            pltpu.VMEM((8, 2, K), jnp.int32),
            pltpu.VMEM((K, D), jnp.float32),
        ] + [pltpu.SemaphoreType.DMA] * 12,
    )(eidx, onesD, zerosD)


def _scatter_body(split_all, table_hbm, eidx_hbm, zeros_hbm, out_hbm,
                  acc, ibuf, gbuf, is0, is1, is2, is3, is4, is5, is6, is7,
                  rs0, rs1, rs2, rs3, ws0, ws1, ws2, ws3):
    c = lax.axis_index("c")
    s = lax.axis_index("s")
    rows = N_ACC // NS
    pltpu.sync_copy(zeros_hbm.at[pl.ds(s * rows, rows)],
                    acc.at[pl.ds(s * rows, rows)])
    plsc.subcore_barrier()
    if split_all:
        n = E_PAD // NW // K
        cb0 = (c * NS + s) * n
        tix = 0
    else:
        n = E_PAD // NS // K
        cb0 = s * n
        tix = c
    isem = (is0, is1, is2, is3, is4, is5, is6, is7)
    rs = (rs0, rs1, rs2, rs3)
    ws = (ws0, ws1, ws2, ws3)

    def i_desc(k, j8):
        return pltpu.make_async_copy(
            eidx_hbm.at[lax.rem(cb0 + k, NCHUNK)], ibuf.at[j8], isem[j8])

    def g_desc(j8):
        j4 = j8 % 4
        return pltpu.make_async_copy(
            table_hbm.at[tix].at[ibuf.at[j8, 0]], gbuf.at[j4], rs[j4])

    def s_desc(j8):
        j4 = j8 % 4
        return pltpu.make_async_copy(
            gbuf.at[j4], acc.at[ibuf.at[j8, 1]], ws[j4])

    def pos(k, j8, wait_w, do_scat):
        if wait_w:
            s_desc((j8 + 4) % 8).wait()
        i_desc(k, j8).wait()
        g_desc(j8).start()
        i_desc(k + 4, (j8 + 4) % 8).start()
        if do_scat:
            g_desc((j8 + 6) % 8).wait()
            s_desc((j8 + 6) % 8).start(add=True)

    for j in range(4):
        i_desc(j, j).start()
    for k in range(8):
        pos(k, k, wait_w=(k >= 4), do_scat=(k >= 2))

    def body(v, carry):
        k0 = 8 * v
        for j in range(8):
            pos(k0 + j, j, wait_w=True, do_scat=True)
        return carry

    lax.fori_loop(1, n // 8, body, 0)
    for j8 in (6, 7):
        g_desc(j8).wait()
        s_desc(j8).start(add=True)
    for j in range(4):
        s_desc(j).wait()
        i_desc(0, j).wait()
    plsc.subcore_barrier()
    pltpu.sync_copy(acc.at[pl.ds(s * rows, rows)],
                    out_hbm.at[c, pl.ds(s * rows, rows)])


def _scatter_partials(table, eidx, zerosD, split_all):
    return pl.kernel(
        functools.partial(_scatter_body, split_all),
        out_type=jax.ShapeDtypeStruct((NC, N_ACC, D), jnp.float32),
        mesh=_mesh(),
        scratch_types=[
            pltpu.VMEM_SHARED((N_ACC, D), jnp.float32),
            pltpu.VMEM((8, 2, K), jnp.int32),
            pltpu.VMEM((4, K, D), jnp.float32),
        ] + [pltpu.SemaphoreType.DMA] * 16,
    )(table, eidx, zerosD)


def _dinv(dega_ref, degb_ref):
    deg = dega_ref[:, 0:1] + degb_ref[:, 0:1] + 1.0
    return lax.rsqrt(deg)


def _h1p_body(x_ref, w_ref, dega_ref, degb_ref, o_ref):
    dinv = _dinv(dega_ref, degb_ref)
    o_ref[...] = jnp.dot(
        x_ref[...], w_ref[...], preferred_element_type=jnp.float32) * dinv


def _h2p_body(s1a_ref, s1b_ref, h1p_ref, dega_ref, degb_ref, b1_ref, w2_ref,
              o_ref):
    dinv = _dinv(dega_ref, degb_ref)
    t = jnp.maximum(
        dinv * (s1a_ref[0] + s1b_ref[0] + h1p_ref[...]) + b1_ref[...],
        0.0)
    o_ref[0] = jnp.dot(
        t, w2_ref[...], preferred_element_type=jnp.float32) * dinv


def _ppi_body(s2_ref, h2p_ref, dega_ref, degb_ref, b2_ref, o_ref):
    dinv = _dinv(dega_ref, degb_ref)
    o_ref[...] = jnp.maximum(
        dinv * (s2_ref[0] + h2p_ref[0]) + b2_ref[0], 0.0)


def _ddi_body(f_ref, w1_ref, b1_ref, w2_ref, b2_ref, w3_ref, b3_ref, o_ref):
    t = jnp.maximum(
        jnp.dot(f_ref[...], w1_ref[...], preferred_element_type=jnp.float32)
        + b1_ref[...], 0.0)
    t = jnp.maximum(
        jnp.dot(t, w2_ref[...], preferred_element_type=jnp.float32)
        + b2_ref[...], 0.0)
    o_ref[...] = jnp.maximum(
        jnp.dot(t, w3_ref[...], preferred_element_type=jnp.float32)
        + b3_ref[...], 0.0)


def kernel(x, edge_index, DDI_features, W1, b1, W2, b2,
           Wf1, bf1, Wf2, bf2, Wf3, bf3):
    row = edge_index[0]
    col = edge_index[1]
    pad = E_PAD - E
    pad_i = jnp.arange(pad, dtype=jnp.int32)
    rowp = jnp.concatenate([row, pad_i % N])
    colp = jnp.concatenate([col, N + pad_i % (N_ACC - N)])
    eidx = jnp.stack(
        [rowp.reshape(NCHUNK, K), colp.reshape(NCHUNK, K)], axis=1)
    onesD = jnp.ones((K, D), jnp.float32)
    zerosD = jnp.zeros((N_ACC, D), jnp.float32)

    deg = _deg_partials(eidx, onesD, zerosD)
    dega, degb = deg[0, :, :8], deg[1, :, :8]

    grid5 = 5
    h1p = pl.pallas_call(
        _h1p_body,
        grid=(grid5,),
        in_specs=[
            pl.BlockSpec((BM, D), lambda i: (i, 0)),
            pl.BlockSpec((D, D), lambda i: (0, 0)),
            pl.BlockSpec((BM, 8), lambda i: (i, 0)),
            pl.BlockSpec((BM, 8), lambda i: (i, 0)),
        ],
        out_specs=pl.BlockSpec((BM, D), lambda i: (i, 0)),
        out_shape=jax.ShapeDtypeStruct((N, D), jnp.float32),
    )(x, W1, dega, degb)

    s1 = _scatter_partials(h1p[None], eidx, zerosD, split_all=True)

    b1r = b1.reshape(1, D)
    h2p = pl.pallas_call(
        _h2p_body,
        grid=(2, grid5),
        in_specs=[
            pl.BlockSpec((1, BM, D), lambda j, i: (0, i, 0)),
            pl.BlockSpec((1, BM, D), lambda j, i: (0, i, 0)),
            pl.BlockSpec((BM, D), lambda j, i: (i, 0)),
            pl.BlockSpec((BM, 8), lambda j, i: (i, 0)),
            pl.BlockSpec((BM, 8), lambda j, i: (i, 0)),
            pl.BlockSpec((1, D), lambda j, i: (0, 0)),
            pl.BlockSpec((D, D), lambda j, i: (0, j)),
        ],
        out_specs=pl.BlockSpec((1, BM, D), lambda j, i: (j, i, 0)),
        out_shape=jax.ShapeDtypeStruct((NC, N, D), jnp.float32),
    )(s1[0:1], s1[1:2], h1p, dega, degb, b1r, W2)

    s2 = _scatter_partials(h2p, eidx, zerosD, split_all=False)

    b2r = b2.reshape(NC, 1, D)
    ppi = pl.pallas_call(
        _ppi_body,
        grid=(2, grid5),
        in_specs=[
            pl.BlockSpec((1, BM, D), lambda j, i: (j, i, 0)),
            pl.BlockSpec((1, BM, D), lambda j, i: (j, i, 0)),
            pl.BlockSpec((BM, 8), lambda j, i: (i, 0)),
            pl.BlockSpec((BM, 8), lambda j, i: (i, 0)),
            pl.BlockSpec((1, 1, D), lambda j, i: (j, 0, 0)),
        ],
        out_specs=pl.BlockSpec((BM, D), lambda j, i: (i, j)),
        out_shape=jax.ShapeDtypeStruct((N, NC * D), jnp.float32),
    )(s2, h2p, dega, degb, b2r)

    B, DDI_DIM = DDI_features.shape
    ddi = pl.pallas_call(
        _ddi_body,
        out_shape=jax.ShapeDtypeStruct((B, 1), jnp.float32),
    )(DDI_features, Wf1, bf1.reshape(1, 64), Wf2, bf2.reshape(1, 16),
      Wf3, bf3.reshape(1, 1))

    return (ppi, ddi)
